# Initial kernel scaffold; baseline (speedup 1.0000x reference)
import jax
import jax.numpy as jnp
from jax.experimental import pallas as pl


def kernel(x, emb, W1, b1, W2, b2):
    raise NotImplementedError("write your pallas kernel here")


# speedup vs baseline: 36.6574x; 36.6574x over previous
import functools

import jax
import jax.numpy as jnp
from jax import lax
from jax.experimental import pallas as pl
from jax.experimental.pallas import tpu as pltpu
from jax.experimental.pallas import tpu_sc as plsc

_NUM_CORES = 2
_NUM_SUBCORES = 16
_NUM_WORKERS = _NUM_CORES * _NUM_SUBCORES
_CHUNK = 128

_TC_ROWS = 8192


def _tc_table_body(emb_ref, w1_ref, b1_ref, w2_ref, b2_ref, out_ref):
    blk = emb_ref[...]
    h = lax.dot_general(w1_ref[...], blk, (((0,), (1,)), ((), ())),
                        preferred_element_type=jnp.float32)
    h = jnp.maximum(h + b1_ref[...], 0.0)
    s = lax.dot_general(w2_ref[...], h, (((0,), (0,)), ((), ())),
                        preferred_element_type=jnp.float32)
    s = jax.nn.sigmoid(s + b2_ref[...])
    out_ref[...] = s[0]


def _make_score_table(emb, W1, b1, W2, b2):
    V, D = emb.shape
    H = W1.shape[1]
    grid = pl.cdiv(V, _TC_ROWS)
    return pl.pallas_call(
        _tc_table_body,
        grid=(grid,),
        in_specs=[
            pl.BlockSpec((_TC_ROWS, D), lambda i: (i, 0)),
            pl.BlockSpec((D, H), lambda i: (0, 0)),
            pl.BlockSpec((H, 1), lambda i: (0, 0)),
            pl.BlockSpec((H, 1), lambda i: (0, 0)),
            pl.BlockSpec((1, 1), lambda i: (0, 0)),
        ],
        out_specs=pl.BlockSpec((_TC_ROWS,), lambda i: (i,)),
        out_shape=jax.ShapeDtypeStruct((V,), jnp.float32),
    )(emb, W1, b1.reshape(H, 1), W2, b2.reshape(1, 1))


def _sc_gather(table, idx3):
    nw, n_chunks, chunk = idx3.shape
    mesh = plsc.VectorSubcoreMesh(
        core_axis_name="c", subcore_axis_name="s",
        num_cores=_NUM_CORES, num_subcores=_NUM_SUBCORES)

    @functools.partial(
        pl.kernel,
        mesh=mesh,
        out_type=jax.ShapeDtypeStruct((nw, n_chunks, chunk), jnp.float32),
        scratch_types=[
            pltpu.VMEM((n_chunks, chunk), jnp.int32),
            pltpu.VMEM((n_chunks, chunk), jnp.float32),
            pltpu.SemaphoreType.DMA,
        ],
    )
    def gather_kernel(table_hbm, idx_hbm, out_hbm, idx_v, rows_v, sem):
        wid = lax.axis_index("s") * _NUM_CORES + lax.axis_index("c")
        pltpu.sync_copy(idx_hbm.at[wid], idx_v)

        def fire(j, carry):
            pltpu.make_async_copy(
                table_hbm.at[idx_v.at[j]], rows_v.at[j], sem).start()
            return carry

        lax.fori_loop(0, n_chunks, fire, 0)

        def drain(j, carry):
            pltpu.make_async_copy(
                table_hbm.at[idx_v.at[j]], rows_v.at[j], sem).wait()
            return carry

        lax.fori_loop(0, n_chunks, drain, 0)
        pltpu.sync_copy(rows_v, out_hbm.at[wid])

    return gather_kernel(table, idx3)


def kernel(x, emb, W1, b1, W2, b2):
    B, L = x.shape
    n = B * L
    assert n % (_NUM_WORKERS * _CHUNK) == 0
    n_chunks = n // (_NUM_WORKERS * _CHUNK)

    table = _make_score_table(emb, W1, b1, W2, b2)
    idx3 = x.reshape(_NUM_WORKERS, n_chunks, _CHUNK)
    scores = _sc_gather(table, idx3)
    return scores.reshape(B, L, 1)


# speedup vs baseline: 36.8224x; 1.0045x over previous
import functools

import jax
import jax.numpy as jnp
from jax import lax
from jax.experimental import pallas as pl
from jax.experimental.pallas import tpu as pltpu
from jax.experimental.pallas import tpu_sc as plsc

_NUM_CORES = 2
_NUM_SUBCORES = 16
_NUM_WORKERS = _NUM_CORES * _NUM_SUBCORES
_CHUNK = 128

_TC_ROWS = 8192


def _tc_table_body(emb_ref, w1_ref, b1_ref, w2_ref, b2_ref, out_ref):
    blk = emb_ref[...]
    h = lax.dot_general(w1_ref[...], blk, (((0,), (1,)), ((), ())),
                        preferred_element_type=jnp.float32)
    h = jnp.maximum(h + b1_ref[...], 0.0)
    s = lax.dot_general(w2_ref[...], h, (((0,), (0,)), ((), ())),
                        preferred_element_type=jnp.float32)
    s = jax.nn.sigmoid(s + b2_ref[...])
    out_ref[...] = s[0]


def _make_score_table(emb, W1, b1, W2, b2):
    V, D = emb.shape
    H = W1.shape[1]
    grid = pl.cdiv(V, _TC_ROWS)
    return pl.pallas_call(
        _tc_table_body,
        grid=(grid,),
        in_specs=[
            pl.BlockSpec((_TC_ROWS, D), lambda i: (i, 0)),
            pl.BlockSpec((D, H), lambda i: (0, 0)),
            pl.BlockSpec((H, 1), lambda i: (0, 0)),
            pl.BlockSpec((H, 1), lambda i: (0, 0)),
            pl.BlockSpec((1, 1), lambda i: (0, 0)),
        ],
        out_specs=pl.BlockSpec((_TC_ROWS,), lambda i: (i,)),
        out_shape=jax.ShapeDtypeStruct((V,), jnp.float32),
    )(emb, W1, b1.reshape(H, 1), W2, b2.reshape(1, 1))


def _sc_gather(table, idx2):
    nw, n_per_w = idx2.shape
    mesh = plsc.VectorSubcoreMesh(
        core_axis_name="c", subcore_axis_name="s",
        num_cores=_NUM_CORES, num_subcores=_NUM_SUBCORES)

    @functools.partial(
        pl.kernel,
        mesh=mesh,
        out_type=jax.ShapeDtypeStruct((nw, n_per_w), jnp.float32),
        scratch_types=[
            pltpu.VMEM((n_per_w,), jnp.int32),
            pltpu.VMEM((n_per_w,), jnp.float32),
            pltpu.SemaphoreType.DMA,
        ],
    )
    def gather_kernel(table_hbm, idx_hbm, out_hbm, idx_v, rows_v, sem):
        wid = lax.axis_index("s") * _NUM_CORES + lax.axis_index("c")
        pltpu.sync_copy(idx_hbm.at[wid], idx_v)
        pltpu.async_copy(table_hbm.at[idx_v], rows_v, sem).wait()
        pltpu.sync_copy(rows_v, out_hbm.at[wid])

    return gather_kernel(table, idx2)


def kernel(x, emb, W1, b1, W2, b2):
    B, L = x.shape
    n = B * L
    assert n % _NUM_WORKERS == 0
    n_per_w = n // _NUM_WORKERS

    table = _make_score_table(emb, W1, b1, W2, b2)
    idx2 = x.reshape(_NUM_WORKERS, n_per_w)
    scores = _sc_gather(table, idx2)
    return scores.reshape(B, L, 1)


# speedup vs baseline: 134.1521x; 3.6432x over previous
import functools

import jax
import jax.numpy as jnp
from jax import lax
from jax.experimental import pallas as pl
from jax.experimental.pallas import tpu as pltpu
from jax.experimental.pallas import tpu_sc as plsc

_NUM_CORES = 2
_NUM_SUBCORES = 16
_NUM_WORKERS = _NUM_CORES * _NUM_SUBCORES
_CHUNK = 128

_TC_COLS = 32768


def _tc_table_body(embt_ref, w1_ref, b1_ref, w2_ref, b2_ref, out_ref):
    blk = embt_ref[...]
    h = lax.dot_general(w1_ref[...], blk, (((0,), (0,)), ((), ())),
                        preferred_element_type=jnp.float32)
    h = jnp.maximum(h + b1_ref[...], 0.0)
    s = lax.dot_general(w2_ref[...], h, (((0,), (0,)), ((), ())),
                        preferred_element_type=jnp.float32)
    s = jax.nn.sigmoid(s + b2_ref[...])
    out_ref[...] = s[0]


def _make_score_table(emb, W1, b1, W2, b2):
    V, D = emb.shape
    H = W1.shape[1]
    embt = emb.T
    grid = pl.cdiv(V, _TC_COLS)
    return pl.pallas_call(
        _tc_table_body,
        grid=(grid,),
        in_specs=[
            pl.BlockSpec((D, _TC_COLS), lambda i: (0, i)),
            pl.BlockSpec((D, H), lambda i: (0, 0)),
            pl.BlockSpec((H, 1), lambda i: (0, 0)),
            pl.BlockSpec((H, 1), lambda i: (0, 0)),
            pl.BlockSpec((1, 1), lambda i: (0, 0)),
        ],
        out_specs=pl.BlockSpec((_TC_COLS,), lambda i: (i,)),
        out_shape=jax.ShapeDtypeStruct((V,), jnp.float32),
    )(embt, W1, b1.reshape(H, 1), W2, b2.reshape(1, 1))


def _sc_gather(table, idx2):
    nw, n_per_w = idx2.shape
    mesh = plsc.VectorSubcoreMesh(
        core_axis_name="c", subcore_axis_name="s",
        num_cores=_NUM_CORES, num_subcores=_NUM_SUBCORES)

    @functools.partial(
        pl.kernel,
        mesh=mesh,
        out_type=jax.ShapeDtypeStruct((nw, n_per_w), jnp.float32),
        scratch_types=[
            pltpu.VMEM((n_per_w,), jnp.int32),
            pltpu.VMEM((n_per_w,), jnp.float32),
            pltpu.SemaphoreType.DMA,
        ],
    )
    def gather_kernel(table_hbm, idx_hbm, out_hbm, idx_v, rows_v, sem):
        wid = lax.axis_index("s") * _NUM_CORES + lax.axis_index("c")
        pltpu.sync_copy(idx_hbm.at[wid], idx_v)
        pltpu.async_copy(table_hbm.at[idx_v], rows_v, sem).wait()
        pltpu.sync_copy(rows_v, out_hbm.at[wid])

    return gather_kernel(table, idx2)


def kernel(x, emb, W1, b1, W2, b2):
    B, L = x.shape
    n = B * L
    assert n % _NUM_WORKERS == 0
    n_per_w = n // _NUM_WORKERS

    table = _make_score_table(emb, W1, b1, W2, b2)
    idx2 = x.reshape(_NUM_WORKERS, n_per_w)
    scores = _sc_gather(table, idx2)
    return scores.reshape(B, L, 1)


# speedup vs baseline: 178.1289x; 1.3278x over previous
import functools

import jax
import jax.numpy as jnp
from jax import lax
from jax.experimental import pallas as pl
from jax.experimental.pallas import tpu as pltpu
from jax.experimental.pallas import tpu_sc as plsc

_NUM_CORES = 2
_NUM_SUBCORES = 16
_NUM_WORKERS = _NUM_CORES * _NUM_SUBCORES
_CHUNK = 128

_TC_COLS = 65536


def _tc_table_body(embt_ref, w1_ref, b1_ref, w2_ref, b2_ref, out_ref):
    blk = embt_ref[...]
    h = lax.dot_general(w1_ref[...], blk, (((0,), (0,)), ((), ())),
                        preferred_element_type=jnp.float32)
    h = jnp.maximum(h + b1_ref[...], 0.0)
    s = lax.dot_general(w2_ref[...], h, (((0,), (0,)), ((), ())),
                        preferred_element_type=jnp.float32)
    s = jax.nn.sigmoid(s + b2_ref[...])
    out_ref[...] = s[0]


def _make_score_table(emb, W1, b1, W2, b2):
    V, D = emb.shape
    H = W1.shape[1]
    embt = emb.T
    grid = pl.cdiv(V, _TC_COLS)
    return pl.pallas_call(
        _tc_table_body,
        grid=(grid,),
        in_specs=[
            pl.BlockSpec((D, _TC_COLS), lambda i: (0, i)),
            pl.BlockSpec((D, H), lambda i: (0, 0)),
            pl.BlockSpec((H, 1), lambda i: (0, 0)),
            pl.BlockSpec((H, 1), lambda i: (0, 0)),
            pl.BlockSpec((1, 1), lambda i: (0, 0)),
        ],
        out_specs=pl.BlockSpec((_TC_COLS,), lambda i: (i,)),
        out_shape=jax.ShapeDtypeStruct((V,), jnp.float32),
    )(embt, W1, b1.reshape(H, 1), W2, b2.reshape(1, 1))


def _sc_gather(table, idx):
    n = idx.shape[0]
    n_per_w = n // _NUM_WORKERS
    mesh = plsc.VectorSubcoreMesh(
        core_axis_name="c", subcore_axis_name="s",
        num_cores=_NUM_CORES, num_subcores=_NUM_SUBCORES)

    @functools.partial(
        pl.kernel,
        mesh=mesh,
        out_type=jax.ShapeDtypeStruct((n,), jnp.float32),
        scratch_types=[
            pltpu.VMEM((n_per_w,), jnp.int32),
            pltpu.VMEM((n_per_w,), jnp.float32),
            pltpu.SemaphoreType.DMA,
        ],
    )
    def gather_kernel(table_hbm, idx_hbm, out_hbm, idx_v, rows_v, sem):
        wid = lax.axis_index("s") * _NUM_CORES + lax.axis_index("c")
        base = wid * n_per_w
        pltpu.sync_copy(idx_hbm.at[pl.ds(base, n_per_w)], idx_v)
        pltpu.async_copy(table_hbm.at[idx_v], rows_v, sem).wait()
        pltpu.sync_copy(rows_v, out_hbm.at[pl.ds(base, n_per_w)])

    return gather_kernel(table, idx)


def kernel(x, emb, W1, b1, W2, b2):
    B, L = x.shape
    n = B * L
    assert n % _NUM_WORKERS == 0

    table = _make_score_table(emb, W1, b1, W2, b2)
    idx = x.T.reshape(n)
    scores = _sc_gather(table, idx)
    return scores.reshape(L, B, 1).transpose(1, 0, 2)
